# baseline (speedup 1.0000x reference)
import jax
import jax.numpy as jnp
from jax import lax
from jax.experimental import pallas as pl
from jax.experimental.pallas import tpu as pltpu
from jax.experimental.pallas import tpu_sc as plsc

DIM = 64
BATCH = 16384
HIST = 50

NC, NS, L = 2, 16, 16
NW = NC * NS
BBLK = 256
SUBLEN = 128
SUB = BBLK // SUBLEN
GROUPS = BBLK // L
NB = BATCH // BBLK
NB_PER_W = NB // NW


def _maxnorm_scale(ss):
    ssc = jnp.maximum(ss, 1.0)
    i = plsc.bitcast(ssc, jnp.int32)
    y = plsc.bitcast(jnp.int32(0x5F3759DF) - (i >> 1), jnp.float32)
    h = 0.5 * ssc
    y = y * (1.5 - h * y * y)
    y = y * (1.5 - h * y * y)
    y = y * (1.5 - h * y * y)
    return jnp.where(ss > 1.0, y, 1.0)


def _body(xt_hbm, table_hbm, out_hbm, idx_v, rows_v, trans_v, gsem):
    wid = lax.axis_index("s") * NC + lax.axis_index("c")
    iota = lax.iota(jnp.int32, L)

    def block_body(blk, carry):
        b0 = (wid * NB_PER_W + blk) * BBLK
        pltpu.sync_copy(xt_hbm.at[:, pl.ds(b0, BBLK)], idx_v)

        def l_body(l, lcarry):
            cps = [
                pltpu.async_copy(
                    table_hbm.at[idx_v.at[l, pl.ds(j * SUBLEN, SUBLEN)]],
                    rows_v.at[pl.ds(j * SUBLEN, SUBLEN)],
                    gsem,
                )
                for j in range(SUB)
            ]
            for cp in cps:
                cp.wait()

            def group_body(g, gcarry):
                rows16 = g * L + iota
                a0 = jnp.zeros((L,), jnp.float32)
                a1 = jnp.zeros((L,), jnp.float32)
                a2 = jnp.zeros((L,), jnp.float32)
                a3 = jnp.zeros((L,), jnp.float32)
                col = iota
                for d in range(0, DIM, 4):
                    v0 = plsc.load_gather(rows_v, [rows16, col])
                    col = (col + 1) & (DIM - 1)
                    v1 = plsc.load_gather(rows_v, [rows16, col])
                    col = (col + 1) & (DIM - 1)
                    v2 = plsc.load_gather(rows_v, [rows16, col])
                    col = (col + 1) & (DIM - 1)
                    v3 = plsc.load_gather(rows_v, [rows16, col])
                    col = (col + 1) & (DIM - 1)
                    a0 = a0 + v0 * v0
                    a1 = a1 + v1 * v1
                    a2 = a2 + v2 * v2
                    a3 = a3 + v3 * v3
                scale = _maxnorm_scale((a0 + a1) + (a2 + a3))
                col = iota
                for d in range(DIM):
                    v = plsc.load_gather(rows_v, [rows16, col])
                    plsc.store_scatter(trans_v, [col, rows16], v * scale)
                    col = (col + 1) & (DIM - 1)
                return gcarry

            lax.fori_loop(0, GROUPS, group_body, 0)
            pltpu.sync_copy(trans_v, out_hbm.at[l, :, pl.ds(b0, BBLK)])
            return lcarry

        lax.fori_loop(0, HIST, l_body, 0)
        return carry

    lax.fori_loop(0, NB_PER_W, block_body, 0)


def kernel(x, table):
    xt = x.T
    mesh = plsc.VectorSubcoreMesh(core_axis_name="c", subcore_axis_name="s")
    out = pl.kernel(
        _body,
        out_type=jax.ShapeDtypeStruct((HIST, DIM, BATCH), jnp.float32),
        mesh=mesh,
        compiler_params=pltpu.CompilerParams(
            use_tc_tiling_on_sc=False, needs_layout_passes=False
        ),
        scratch_types=[
            pltpu.VMEM((HIST, BBLK), jnp.int32),
            pltpu.VMEM((BBLK, DIM), jnp.float32),
            pltpu.VMEM((DIM, BBLK), jnp.float32),
            pltpu.SemaphoreType.DMA,
        ],
    )(xt, table)
    return out.transpose(2, 0, 1)


# speedup vs baseline: 1.1133x; 1.1133x over previous
import jax
import jax.numpy as jnp
from jax import lax
from jax.experimental import pallas as pl
from jax.experimental.pallas import tpu as pltpu
from jax.experimental.pallas import tpu_sc as plsc

DIM = 64
BATCH = 16384
HIST = 50

NC, NS, L = 2, 16, 16
NW = NC * NS
BBLK = 256
SUBLEN = 128
SUB = BBLK // SUBLEN
GROUPS = BBLK // L
NB = BATCH // BBLK
NB_PER_W = NB // NW
NCHUNK = NB_PER_W * HIST


def _maxnorm_scale(ss):
    ssc = jnp.maximum(ss, 1.0)
    i = plsc.bitcast(ssc, jnp.int32)
    y = plsc.bitcast(jnp.int32(0x5F3759DF) - (i >> 1), jnp.float32)
    h = 0.5 * ssc
    y = y * (1.5 - h * y * y)
    y = y * (1.5 - h * y * y)
    y = y * (1.5 - h * y * y)
    return jnp.where(ss > 1.0, y, 1.0)


def _body(xt_hbm, table_hbm, out_hbm, idx_v, rows_v, trans_v,
          gsem0, gsem1, wsem0, wsem1):
    wid = lax.axis_index("s") * NC + lax.axis_index("c")
    iota = lax.iota(jnp.int32, L)
    gsems = (gsem0, gsem1)
    wsems = (wsem0, wsem1)

    for blk in range(NB_PER_W):
        pltpu.sync_copy(
            xt_hbm.at[:, pl.ds((wid * NB_PER_W + blk) * BBLK, BBLK)],
            idx_v.at[blk],
        )

    def chunk_coords(i):
        blk = jnp.where(i >= HIST, 1, 0)
        l = i - blk * HIST
        b0 = (wid * NB_PER_W + blk) * BBLK
        return blk, l, b0

    def issue_gather(i, slot):
        blk, l, _ = chunk_coords(i)
        for j in range(SUB):
            pltpu.async_copy(
                table_hbm.at[idx_v.at[blk, l, pl.ds(j * SUBLEN, SUBLEN)]],
                rows_v.at[slot, pl.ds(j * SUBLEN, SUBLEN)],
                gsems[slot],
            )

    def compute(slot):
        rows = rows_v.at[slot]
        trans = trans_v.at[slot]

        def group_body(g, gcarry):
            rows16 = g * L + iota
            a0 = jnp.zeros((L,), jnp.float32)
            a1 = jnp.zeros((L,), jnp.float32)
            a2 = jnp.zeros((L,), jnp.float32)
            a3 = jnp.zeros((L,), jnp.float32)
            col = iota
            for d in range(0, DIM, 4):
                v0 = plsc.load_gather(rows, [rows16, col])
                col = (col + 1) & (DIM - 1)
                v1 = plsc.load_gather(rows, [rows16, col])
                col = (col + 1) & (DIM - 1)
                v2 = plsc.load_gather(rows, [rows16, col])
                col = (col + 1) & (DIM - 1)
                v3 = plsc.load_gather(rows, [rows16, col])
                col = (col + 1) & (DIM - 1)
                a0 = a0 + v0 * v0
                a1 = a1 + v1 * v1
                a2 = a2 + v2 * v2
                a3 = a3 + v3 * v3
            scale = _maxnorm_scale((a0 + a1) + (a2 + a3))
            col = iota
            for d in range(DIM):
                v = plsc.load_gather(rows, [rows16, col])
                plsc.store_scatter(trans, [col, rows16], v * scale)
                col = (col + 1) & (DIM - 1)
            return gcarry

        lax.fori_loop(0, GROUPS, group_body, 0)

    issue_gather(0, 0)

    def outer(i2, carry):
        for s in range(2):
            i = i2 * 2 + s
            cur, nxt = s, 1 - s
            pltpu.make_async_copy(
                table_hbm.at[pl.ds(0, BBLK)], rows_v.at[cur], gsems[cur]
            ).wait()
            @pl.when(i + 1 < NCHUNK)
            def _():
                issue_gather(i + 1, nxt)
            @pl.when(i >= 2)
            def _():
                pltpu.make_async_copy(
                    trans_v.at[cur],
                    out_hbm.at[0, :, pl.ds(0, BBLK)],
                    wsems[cur],
                ).wait()
            compute(cur)
            _, l, b0 = chunk_coords(i)
            pltpu.async_copy(
                trans_v.at[cur],
                out_hbm.at[l, :, pl.ds(b0, BBLK)],
                wsems[cur],
            )
        return carry

    lax.fori_loop(0, NCHUNK // 2, outer, 0)

    for s in range(2):
        pltpu.make_async_copy(
            trans_v.at[s], out_hbm.at[0, :, pl.ds(0, BBLK)], wsems[s]
        ).wait()


def kernel(x, table):
    xt = x.T
    mesh = plsc.VectorSubcoreMesh(core_axis_name="c", subcore_axis_name="s")
    out = pl.kernel(
        _body,
        out_type=jax.ShapeDtypeStruct((HIST, DIM, BATCH), jnp.float32),
        mesh=mesh,
        compiler_params=pltpu.CompilerParams(
            use_tc_tiling_on_sc=False, needs_layout_passes=False
        ),
        scratch_types=[
            pltpu.VMEM((NB_PER_W, HIST, BBLK), jnp.int32),
            pltpu.VMEM((2, BBLK, DIM), jnp.float32),
            pltpu.VMEM((2, DIM, BBLK), jnp.float32),
            pltpu.SemaphoreType.DMA,
            pltpu.SemaphoreType.DMA,
            pltpu.SemaphoreType.DMA,
            pltpu.SemaphoreType.DMA,
        ],
    )(xt, table)
    return out.transpose(2, 0, 1)


# speedup vs baseline: 1.5298x; 1.3741x over previous
import jax
import jax.numpy as jnp
from jax import lax
from jax.experimental import pallas as pl
from jax.experimental.pallas import tpu as pltpu
from jax.experimental.pallas import tpu_sc as plsc

DIM = 64
BATCH = 16384
HIST = 50

NC, NS, L = 2, 16, 16
NW = NC * NS
BBLK = 512
SUBLEN = 128
SUB = BBLK // SUBLEN
GROUPS = BBLK // L
NCHUNK = HIST

_GDN = lax.GatherDimensionNumbers(
    offset_dims=(), collapsed_slice_dims=(0,), start_index_map=(0,)
)


def _lane_shuffle(v, idx):
    return lax.gather(
        v, idx[:, None], _GDN, slice_sizes=(1,),
        mode=lax.GatherScatterMode.PROMISE_IN_BOUNDS,
    )


def _maxnorm_scale(ss):
    ssc = jnp.maximum(ss, 1.0)
    i = plsc.bitcast(ssc, jnp.int32)
    y = plsc.bitcast(jnp.int32(0x5F3759DF) - (i >> 1), jnp.float32)
    h = 0.5 * ssc
    y = y * (1.5 - h * y * y)
    y = y * (1.5 - h * y * y)
    y = y * (1.5 - h * y * y)
    return jnp.where(ss > 1.0, y, 1.0)


def _body(xt_hbm, table_hbm, out_hbm, idx_v, rows_v, gsem0, gsem1,
          wsem0, wsem1):
    wid = lax.axis_index("s") * NC + lax.axis_index("c")
    iota = lax.iota(jnp.int32, L)
    gsems = (gsem0, gsem1)
    wsems = (wsem0, wsem1)
    b0 = wid * BBLK

    pltpu.sync_copy(xt_hbm.at[:, pl.ds(b0, BBLK)], idx_v)

    def issue_gather(l, slot):
        for j in range(SUB):
            pltpu.async_copy(
                table_hbm.at[idx_v.at[l, pl.ds(j * SUBLEN, SUBLEN)]],
                rows_v.at[slot, pl.ds(j * SUBLEN, SUBLEN)],
                gsems[slot],
            )

    def compute(slot):
        rows = rows_v.at[slot]

        def group_body(g, gcarry):
            r0 = g * L
            packed = jnp.zeros((L,), jnp.float32)
            for rr in range(L):
                r = r0 + rr
                v0 = rows[r, pl.ds(0, L)]
                v1 = rows[r, pl.ds(L, L)]
                v2 = rows[r, pl.ds(2 * L, L)]
                v3 = rows[r, pl.ds(3 * L, L)]
                ss = (v0 * v0 + v1 * v1) + (v2 * v2 + v3 * v3)
                for m in (8, 4, 2, 1):
                    ss = ss + _lane_shuffle(ss, iota ^ m)
                packed = jnp.where(iota == rr, ss, packed)
            scale = _maxnorm_scale(packed)
            for rr in range(L):
                r = r0 + rr
                sc = _lane_shuffle(scale, jnp.full((L,), rr, jnp.int32))
                for k in range(4):
                    sl = pl.ds(k * L, L)
                    rows[r, sl] = rows[r, sl] * sc
            return gcarry

        lax.fori_loop(0, GROUPS, group_body, 0)

    issue_gather(0, 0)

    def outer(i2, carry):
        for s in range(2):
            i = i2 * 2 + s
            cur, nxt = s, 1 - s
            pltpu.make_async_copy(
                table_hbm.at[pl.ds(0, BBLK)], rows_v.at[cur], gsems[cur]
            ).wait()
            @pl.when(i + 1 < NCHUNK)
            def _():
                @pl.when(i >= 1)
                def _():
                    pltpu.make_async_copy(
                        rows_v.at[nxt],
                        out_hbm.at[0, pl.ds(0, BBLK), :],
                        wsems[nxt],
                    ).wait()
                issue_gather(i + 1, nxt)
            compute(cur)
            pltpu.async_copy(
                rows_v.at[cur],
                out_hbm.at[i, pl.ds(b0, BBLK), :],
                wsems[cur],
            )
        return carry

    lax.fori_loop(0, NCHUNK // 2, outer, 0)

    for s in range(2):
        pltpu.make_async_copy(
            rows_v.at[s], out_hbm.at[0, pl.ds(0, BBLK), :], wsems[s]
        ).wait()


def kernel(x, table):
    xt = x.T
    mesh = plsc.VectorSubcoreMesh(core_axis_name="c", subcore_axis_name="s")
    out = pl.kernel(
        _body,
        out_type=jax.ShapeDtypeStruct((HIST, BATCH, DIM), jnp.float32),
        mesh=mesh,
        compiler_params=pltpu.CompilerParams(
            use_tc_tiling_on_sc=False, needs_layout_passes=False
        ),
        scratch_types=[
            pltpu.VMEM((HIST, BBLK), jnp.int32),
            pltpu.VMEM((2, BBLK, DIM), jnp.float32),
            pltpu.SemaphoreType.DMA,
            pltpu.SemaphoreType.DMA,
            pltpu.SemaphoreType.DMA,
            pltpu.SemaphoreType.DMA,
        ],
    )(xt, table)
    return out.transpose(1, 0, 2)
